# Initial kernel scaffold; baseline (speedup 1.0000x reference)
import jax
import jax.numpy as jnp
from jax.experimental import pallas as pl


def kernel(x, edge_index, W1_l, b1_l, W1_r, W2_l, b2_l, W2_r):
    raise NotImplementedError("write your pallas kernel here")


# speedup vs baseline: 16.3452x; 16.3452x over previous
import functools

import jax
import jax.numpy as jnp
from jax import lax
from jax.experimental import pallas as pl
from jax.experimental.pallas import tpu as pltpu
from jax.experimental.pallas import tpu_sc as plsc

N = 10000
E = 320000
IN_CH = 128
HID = 32

NC = 2
NS = 16
NW = NC * NS
EPW = E // NW
CH = 1000
NCHUNK = EPW // CH
RA = 624
RLAST = N - (NS - 1) * RA
CNTW = 16


def _f32(*shape):
    return jax.ShapeDtypeStruct(shape, jnp.float32)


def _sc_pass(src, dst, y, with_cnt):
    mesh = plsc.VectorSubcoreMesh(
        core_axis_name="c", subcore_axis_name="s", num_cores=NC, num_subcores=NS
    )

    out_type = [_f32(NC, N, HID)]
    if with_cnt:
        out_type.append(_f32(NC, N, CNTW))

    scratch = [
        pltpu.VMEM((CH,), jnp.int32),
        pltpu.VMEM((CH,), jnp.int32),
        pltpu.VMEM((CH, HID), jnp.float32),
        pltpu.VMEM((RLAST, HID), jnp.float32),
        pltpu.SemaphoreType.DMA,
        pltpu.VMEM_SHARED((N, HID), jnp.float32),
    ]
    if with_cnt:
        scratch += [
            pltpu.VMEM((CH, CNTW), jnp.float32),
            pltpu.VMEM((RLAST, CNTW), jnp.float32),
            pltpu.VMEM_SHARED((N, CNTW), jnp.float32),
        ]

    def body(src_hbm, dst_hbm, y_hbm, *rest):
        if with_cnt:
            (acc_out, cnt_out, src_v, dst_v, rows_v, zrow_v, sem, acc_sh,
             ones_v, zcnt_v, cnt_sh) = rest
        else:
            acc_out, src_v, dst_v, rows_v, zrow_v, sem, acc_sh = rest

        cid = lax.axis_index("c")
        sid = lax.axis_index("s")
        wid = sid * NC + cid
        row0 = pl.multiple_of(sid * RA, 8)
        is_last = sid == NS - 1

        def zero_buf(buf, ncols):
            def zb(i, _):
                r = i // (ncols // 16)
                c = (i % (ncols // 16)) * 16
                buf[r, pl.ds(c, 16)] = jnp.zeros((16,), jnp.float32)
                return 0
            lax.fori_loop(0, RLAST * (ncols // 16), zb, 0)

        def zero_shared(buf, sh_ref):
            @pl.when(is_last)
            def _():
                pltpu.sync_copy(buf, sh_ref.at[pl.ds(row0, RLAST)])

            @pl.when(jnp.logical_not(is_last))
            def _():
                pltpu.sync_copy(buf.at[pl.ds(0, RA)], sh_ref.at[pl.ds(row0, RA)])

        zero_buf(zrow_v, HID)
        zero_shared(zrow_v, acc_sh)

        if with_cnt:
            zero_buf(zcnt_v, CNTW)
            zero_shared(zcnt_v, cnt_sh)

            def fill_ones(i, _):
                ones_v[i, pl.ds(0, 16)] = jnp.ones((16,), jnp.float32)
                return 0

            lax.fori_loop(0, CH, fill_ones, 0)

        plsc.subcore_barrier()

        def chunk_body(g, _):
            base = wid * EPW + g * CH
            pltpu.sync_copy(src_hbm.at[pl.ds(base, CH)], src_v)
            pltpu.sync_copy(dst_hbm.at[pl.ds(base, CH)], dst_v)
            pltpu.async_copy(y_hbm.at[src_v], rows_v, sem).wait()
            pltpu.sync_copy(rows_v, acc_sh.at[dst_v], add=True)
            if with_cnt:
                pltpu.sync_copy(ones_v, cnt_sh.at[dst_v], add=True)
            return 0

        lax.fori_loop(0, NCHUNK, chunk_body, 0)
        plsc.subcore_barrier()

        def copy_out(sh_ref, out_ref):
            @pl.when(is_last)
            def _():
                pltpu.sync_copy(sh_ref.at[pl.ds(row0, RLAST)],
                                out_ref.at[cid, pl.ds(row0, RLAST)])

            @pl.when(jnp.logical_not(is_last))
            def _():
                pltpu.sync_copy(sh_ref.at[pl.ds(row0, RA)],
                                out_ref.at[cid, pl.ds(row0, RA)])

        copy_out(acc_sh, acc_out)
        if with_cnt:
            copy_out(cnt_sh, cnt_out)

    run = pl.kernel(
        body, out_type=out_type, mesh=mesh, scratch_types=scratch,
        compiler_params=pltpu.CompilerParams(use_tc_tiling_on_sc=False),
        name="sage_sc_pass",
    )
    res = run(src, dst, y)
    if with_cnt:
        return res[0], res[1]
    return res[0], None


def _dotT(a, w):
    return lax.dot_general(a, w, (((1,), (1,)), ((), ())),
                           preferred_element_type=jnp.float32)


def _pre_body(x_ref, wl_ref, wr_ref, y1_ref, xr_ref):
    x = x_ref[...]
    y1_ref[...] = _dotT(x, wl_ref[...])
    xr_ref[...] = _dotT(x, wr_ref[...])


def _mid_body(acc_ref, cntp_ref, xr_ref, b1_ref, w2l_ref, w2r_ref,
              y2_ref, hr_ref, cnt_ref):
    acc = acc_ref[0] + acc_ref[1]
    cnt16 = jnp.clip(cntp_ref[0] + cntp_ref[1], 1.0, None)
    cnt = cnt16[:, 0:1]
    h = jnp.maximum(acc / cnt + b1_ref[...] + xr_ref[...], 0.0)
    y2_ref[...] = _dotT(h, w2l_ref[...])
    hr_ref[...] = _dotT(h, w2r_ref[...])
    cnt_ref[...] = cnt16


def _out_body(acc_ref, cnt_ref, hr_ref, b2_ref, out_ref):
    acc = acc_ref[0] + acc_ref[1]
    cnt = cnt_ref[:, 0:1]
    out_ref[...] = acc / cnt + b2_ref[...] + hr_ref[...]


def kernel(x, edge_index, W1_l, b1_l, W1_r, W2_l, b2_l, W2_r):
    src = edge_index[0]
    dst = edge_index[1]

    y1, xr = pl.pallas_call(
        _pre_body,
        out_shape=[_f32(N, HID), _f32(N, HID)],
    )(x, W1_l, W1_r)

    acc1, cntp = _sc_pass(src, dst, y1, with_cnt=True)

    y2, hr, cnt = pl.pallas_call(
        _mid_body,
        out_shape=[_f32(N, HID), _f32(N, HID), _f32(N, CNTW)],
    )(acc1, cntp, xr, b1_l.reshape(1, HID), W2_l, W2_r)

    acc2, _ = _sc_pass(src, dst, y2, with_cnt=False)

    out = pl.pallas_call(
        _out_body,
        out_shape=_f32(N, HID),
    )(acc2, cnt, hr, b2_l.reshape(1, HID))

    return out


# speedup vs baseline: 19.6056x; 1.1995x over previous
import functools

import jax
import jax.numpy as jnp
from jax import lax
from jax.experimental import pallas as pl
from jax.experimental.pallas import tpu as pltpu
from jax.experimental.pallas import tpu_sc as plsc

N = 10000
E = 320000
IN_CH = 128
HID = 32

NC = 2
NS = 16
NW = NC * NS
EPW = E // NW
CH = 1000
NCHUNK = EPW // CH
RA = 624
RLAST = N - (NS - 1) * RA
CNTW = 16


def _f32(*shape):
    return jax.ShapeDtypeStruct(shape, jnp.float32)


def _sc_pass(src, dst, y, with_cnt):
    mesh = plsc.VectorSubcoreMesh(
        core_axis_name="c", subcore_axis_name="s", num_cores=NC, num_subcores=NS
    )

    out_type = [_f32(NC, N, HID)]
    if with_cnt:
        out_type.append(_f32(NC, N, CNTW))

    scratch = [
        pltpu.VMEM((CH,), jnp.int32),
        pltpu.VMEM((CH,), jnp.int32),
        pltpu.VMEM((CH,), jnp.int32),
        pltpu.VMEM((CH,), jnp.int32),
        pltpu.VMEM((CH, HID), jnp.float32),
        pltpu.VMEM((CH, HID), jnp.float32),
        pltpu.SemaphoreType.DMA,
        pltpu.SemaphoreType.DMA,
        pltpu.VMEM_SHARED((N, HID), jnp.float32),
    ]
    if with_cnt:
        scratch += [
            pltpu.VMEM((CH, CNTW), jnp.float32),
            pltpu.VMEM_SHARED((N, CNTW), jnp.float32),
        ]

    def body(src_hbm, dst_hbm, y_hbm, *rest):
        if with_cnt:
            (acc_out, cnt_out, src0, src1, dst0, dst1, rows0, rows1,
             sem0, sem1, acc_sh, ones_v, cnt_sh) = rest
        else:
            (acc_out, src0, src1, dst0, dst1, rows0, rows1,
             sem0, sem1, acc_sh) = rest

        cid = lax.axis_index("c")
        sid = lax.axis_index("s")
        wid = sid * NC + cid
        row0 = pl.multiple_of(sid * RA, 8)
        is_last = sid == NS - 1

        def zero_buf(buf, ncols):
            def zb(i, _):
                for j in range(ncols // 16):
                    buf[i, pl.ds(j * 16, 16)] = jnp.zeros((16,), jnp.float32)
                return 0
            lax.fori_loop(0, RLAST, zb, 0)

        def zero_shared(buf, sh_ref):
            @pl.when(is_last)
            def _():
                pltpu.sync_copy(buf.at[pl.ds(0, RLAST)],
                                sh_ref.at[pl.ds(row0, RLAST)])

            @pl.when(jnp.logical_not(is_last))
            def _():
                pltpu.sync_copy(buf.at[pl.ds(0, RA)], sh_ref.at[pl.ds(row0, RA)])

        zero_buf(rows0, HID)
        zero_shared(rows0, acc_sh)

        if with_cnt:
            zero_buf(ones_v, CNTW)
            zero_shared(ones_v, cnt_sh)

            def fill_ones(i, _):
                ones_v[i, pl.ds(0, 16)] = jnp.ones((16,), jnp.float32)
                return 0

            lax.fori_loop(0, CH, fill_ones, 0)

        plsc.subcore_barrier()

        def load_idx(g, sv, dv):
            base = pl.multiple_of(wid * EPW + g * CH, 8)
            pltpu.sync_copy(src_hbm.at[pl.ds(base, CH)], sv)
            pltpu.sync_copy(dst_hbm.at[pl.ds(base, CH)], dv)

        bufs = ((src0, dst0, rows0, sem0), (src1, dst1, rows1, sem1))

        load_idx(0, src0, dst0)
        pltpu.make_async_copy(y_hbm.at[src0], rows0, sem0).start()

        def pair_body(i, _):
            for b in range(2):
                g = 2 * i + b
                sv, dv, rv, sm = bufs[b]
                sv2, dv2, rv2, sm2 = bufs[1 - b]

                def prefetch():
                    load_idx(g + 1, sv2, dv2)
                    pltpu.make_async_copy(y_hbm.at[sv2], rv2, sm2).start()

                if b == 0:
                    prefetch()
                else:
                    @pl.when(i < NCHUNK // 2 - 1)
                    def _():
                        prefetch()

                pltpu.make_async_copy(y_hbm.at[sv], rv, sm).wait()
                pltpu.sync_copy(rv, acc_sh.at[dv], add=True)
                if with_cnt:
                    pltpu.sync_copy(ones_v, cnt_sh.at[dv], add=True)
            return 0

        lax.fori_loop(0, NCHUNK // 2, pair_body, 0)
        plsc.subcore_barrier()

        def copy_out(sh_ref, out_ref):
            @pl.when(is_last)
            def _():
                pltpu.sync_copy(sh_ref.at[pl.ds(row0, RLAST)],
                                out_ref.at[cid, pl.ds(row0, RLAST)])

            @pl.when(jnp.logical_not(is_last))
            def _():
                pltpu.sync_copy(sh_ref.at[pl.ds(row0, RA)],
                                out_ref.at[cid, pl.ds(row0, RA)])

        copy_out(acc_sh, acc_out)
        if with_cnt:
            copy_out(cnt_sh, cnt_out)

    run = pl.kernel(
        body, out_type=out_type, mesh=mesh, scratch_types=scratch,
        compiler_params=pltpu.CompilerParams(use_tc_tiling_on_sc=False),
        name="sage_sc_pass",
    )
    res = run(src, dst, y)
    if with_cnt:
        return res[0], res[1]
    return res[0], None


def _dotT(a, w):
    return lax.dot_general(a, w, (((1,), (1,)), ((), ())),
                           preferred_element_type=jnp.float32)


def _pre_body(x_ref, wl_ref, wr_ref, y1_ref, xr_ref):
    x = x_ref[...]
    y1_ref[...] = _dotT(x, wl_ref[...])
    xr_ref[...] = _dotT(x, wr_ref[...])


def _mid_body(acc_ref, cntp_ref, xr_ref, b1_ref, w2l_ref, w2r_ref,
              y2_ref, hr_ref, cnt_ref):
    acc = acc_ref[0] + acc_ref[1]
    cnt16 = jnp.clip(cntp_ref[0] + cntp_ref[1], 1.0, None)
    cnt = cnt16[:, 0:1]
    h = jnp.maximum(acc / cnt + b1_ref[...] + xr_ref[...], 0.0)
    y2_ref[...] = _dotT(h, w2l_ref[...])
    hr_ref[...] = _dotT(h, w2r_ref[...])
    cnt_ref[...] = cnt16


def _out_body(acc_ref, cnt_ref, hr_ref, b2_ref, out_ref):
    acc = acc_ref[0] + acc_ref[1]
    cnt = cnt_ref[:, 0:1]
    out_ref[...] = acc / cnt + b2_ref[...] + hr_ref[...]


def kernel(x, edge_index, W1_l, b1_l, W1_r, W2_l, b2_l, W2_r):
    src = edge_index[0]
    dst = edge_index[1]

    y1, xr = pl.pallas_call(
        _pre_body,
        out_shape=[_f32(N, HID), _f32(N, HID)],
    )(x, W1_l, W1_r)

    acc1, cntp = _sc_pass(src, dst, y1, with_cnt=True)

    y2, hr, cnt = pl.pallas_call(
        _mid_body,
        out_shape=[_f32(N, HID), _f32(N, HID), _f32(N, CNTW)],
    )(acc1, cntp, xr, b1_l.reshape(1, HID), W2_l, W2_r)

    acc2, _ = _sc_pass(src, dst, y2, with_cnt=False)

    out = pl.pallas_call(
        _out_body,
        out_shape=_f32(N, HID),
    )(acc2, cnt, hr, b2_l.reshape(1, HID))

    return out
